# baseline (speedup 1.0000x reference)
import jax
import jax.numpy as jnp
from jax import lax
from jax.experimental import pallas as pl
from jax.experimental.pallas import tpu as pltpu
from jax.experimental.pallas import tpu_sc as plsc

_NC = 2
_NS = 16
_NW = _NC * _NS
_L = 16

_N = 4 * 8192
_D = 32
_H = 16
_T = _N // _NW
_G = 2


def _sc_body(emb_hbm, ws_hbm, b1s_hbm, w2s_hbm, b2s_hbm, out_hbm,
             emb_v, ws_v, b1s_v, w2s_v, b2s_v, out_v):
    wid = lax.axis_index("s") * _NC + lax.axis_index("c")
    base = wid * _T
    pltpu.sync_copy(emb_hbm.at[pl.ds(base * _D, _T * _D)], emb_v)
    pltpu.sync_copy(ws_hbm, ws_v)
    pltpu.sync_copy(b1s_hbm, b1s_v)
    pltpu.sync_copy(w2s_hbm, w2s_v)
    pltpu.sync_copy(b2s_hbm, b2s_v)

    b2vec = b2s_v[...]
    lane = lax.iota(jnp.int32, _L) * _D

    @plsc.parallel_loop(0, _T // (_L * _G), unroll=2)
    def body(i):
        t0 = i * (_L * _G)
        idx = [lane + (t0 + g * _L) * _D for g in range(_G)]
        hs = [[b1s_v[pl.ds(_L * j, _L)] for g in range(_G)] for j in range(_H)]
        for d in range(_D):
            xs = [plsc.load_gather(emb_v, [idx[g] + d]) for g in range(_G)]
            for j in range(_H):
                w = ws_v[pl.ds((j * _D + d) * _L, _L)]
                for g in range(_G):
                    hs[j][g] = hs[j][g] + xs[g] * w
        zs = [b2vec for g in range(_G)]
        for j in range(_H):
            w2 = w2s_v[pl.ds(_L * j, _L)]
            for g in range(_G):
                zs[g] = zs[g] + jnp.maximum(hs[j][g], 0.0) * w2
        for g in range(_G):
            s = 1.0 / (1.0 + jnp.exp(-zs[g]))
            out_v[pl.ds(t0 + g * _L, _L)] = s

    pltpu.sync_copy(out_v, out_hbm.at[pl.ds(base, _T)])


@jax.jit
def _run(flat_emb, ws, b1s, w2s, b2s):
    mesh = plsc.VectorSubcoreMesh(core_axis_name="c", subcore_axis_name="s")
    return pl.kernel(
        _sc_body,
        out_type=jax.ShapeDtypeStruct((_N,), jnp.float32),
        mesh=mesh,
        compiler_params=pltpu.CompilerParams(needs_layout_passes=False),
        scratch_types=[
            pltpu.VMEM((_T * _D,), jnp.float32),
            pltpu.VMEM((_H * _D * _L,), jnp.float32),
            pltpu.VMEM((_H * _L,), jnp.float32),
            pltpu.VMEM((_H * _L,), jnp.float32),
            pltpu.VMEM((_L,), jnp.float32),
            pltpu.VMEM((_T,), jnp.float32),
        ],
    )(flat_emb, ws, b1s, w2s, b2s)


def kernel(embeddings, W1, b1, W2, b2):
    bsz, seq, _ = embeddings.shape
    flat = embeddings.reshape(-1)
    ws = jnp.repeat(W1.reshape(-1), _L)
    b1s = jnp.repeat(b1, _L)
    w2s = jnp.repeat(W2.reshape(-1), _L)
    b2s = jnp.broadcast_to(b2, (_L,)).astype(jnp.float32)
    out = _run(flat, ws, b1s, w2s, b2s)
    return out.reshape(bsz, seq)


# speedup vs baseline: 2.6720x; 2.6720x over previous
import jax
import jax.numpy as jnp
from jax.experimental import pallas as pl
from jax.experimental.pallas import tpu as pltpu

_N = 4 * 8192
_D = 32
_H = 16
_BLK = 4096


def _tc_body(x_ref, w1t_ref, b1_ref, w2_ref, b2_ref, o_ref):
    x = x_ref[...]
    h = jnp.dot(x, w1t_ref[...], preferred_element_type=jnp.float32)
    h = jnp.maximum(h + b1_ref[...], 0.0)
    z = jnp.sum(h * w2_ref[...], axis=1) + b2_ref[0]
    o_ref[...] = 1.0 / (1.0 + jnp.exp(-z))


@jax.jit
def _run_tc(flat2d, w1t, b1, w2, b2):
    return pl.pallas_call(
        _tc_body,
        out_shape=jax.ShapeDtypeStruct((_N,), jnp.float32),
        grid=(_N // _BLK,),
        in_specs=[
            pl.BlockSpec((_BLK, _D), lambda i: (i, 0)),
            pl.BlockSpec((_D, _H), lambda i: (0, 0)),
            pl.BlockSpec((1, _H), lambda i: (0, 0)),
            pl.BlockSpec((1, _H), lambda i: (0, 0)),
            pl.BlockSpec(memory_space=pltpu.SMEM),
        ],
        out_specs=pl.BlockSpec((_BLK,), lambda i: (i,)),
    )(flat2d, w1t, b1, w2, b2)


def kernel(embeddings, W1, b1, W2, b2):
    bsz, seq, _ = embeddings.shape
    flat = embeddings.reshape(-1, _D)
    out = _run_tc(flat, W1.T, b1.reshape(1, _H), W2.reshape(1, _H), b2)
    return out.reshape(bsz, seq)


# speedup vs baseline: 3.7619x; 1.4079x over previous
import jax
import jax.numpy as jnp
from jax import lax
from jax.experimental import pallas as pl
from jax.experimental.pallas import tpu as pltpu

_N = 4 * 8192
_D = 32
_H = 16
_BLK = 4096


def _tc_body(x_ref, w1_ref, b1_ref, w2_ref, b2_ref, o_ref):
    x = x_ref[...]
    ht = lax.dot_general(w1_ref[...], x, (((1,), (1,)), ((), ())),
                         preferred_element_type=jnp.float32)
    ht = jnp.maximum(ht + b1_ref[...], 0.0)
    zt = lax.dot_general(w2_ref[...], ht, (((1,), (0,)), ((), ())),
                         preferred_element_type=jnp.float32)
    z = zt + b2_ref[0]
    o_ref[...] = 1.0 / (1.0 + jnp.exp(-z))


@jax.jit
def _run_tc(flat2d, w1, b1, w2, b2):
    return pl.pallas_call(
        _tc_body,
        out_shape=jax.ShapeDtypeStruct((1, _N), jnp.float32),
        grid=(_N // _BLK,),
        in_specs=[
            pl.BlockSpec((_BLK, _D), lambda i: (i, 0)),
            pl.BlockSpec((_H, _D), lambda i: (0, 0)),
            pl.BlockSpec((_H, 1), lambda i: (0, 0)),
            pl.BlockSpec((1, _H), lambda i: (0, 0)),
            pl.BlockSpec(memory_space=pltpu.SMEM),
        ],
        out_specs=pl.BlockSpec((1, _BLK), lambda i: (0, i)),
    )(flat2d, w1, b1, w2, b2)


def kernel(embeddings, W1, b1, W2, b2):
    bsz, seq, _ = embeddings.shape
    flat = embeddings.reshape(-1, _D)
    out = _run_tc(flat, W1, b1.reshape(_H, 1), W2.reshape(1, _H), b2)
    return out.reshape(bsz, seq)


# speedup vs baseline: 4.1046x; 1.0911x over previous
import jax
import jax.numpy as jnp
from jax import lax
from jax.experimental import pallas as pl
from jax.experimental.pallas import tpu as pltpu

_N = 4 * 8192
_D = 32
_H = 16
_BLK = 8192


def _tc_body(x_ref, w1_ref, b1_ref, w2_ref, b2_ref, o_ref):
    x = x_ref[...]
    ht = lax.dot_general(w1_ref[...], x, (((1,), (1,)), ((), ())),
                         preferred_element_type=jnp.float32)
    ht = jnp.maximum(ht + b1_ref[...], 0.0)
    zt = lax.dot_general(w2_ref[...], ht, (((1,), (0,)), ((), ())),
                         preferred_element_type=jnp.float32)
    z = zt + b2_ref[0]
    o_ref[...] = 1.0 / (1.0 + jnp.exp(-z))


@jax.jit
def _run_tc(flat2d, w1, b1, w2, b2):
    return pl.pallas_call(
        _tc_body,
        out_shape=jax.ShapeDtypeStruct((1, _N), jnp.float32),
        grid=(_N // _BLK,),
        in_specs=[
            pl.BlockSpec((_BLK, _D), lambda i: (i, 0)),
            pl.BlockSpec((_H, _D), lambda i: (0, 0)),
            pl.BlockSpec((_H, 1), lambda i: (0, 0)),
            pl.BlockSpec((1, _H), lambda i: (0, 0)),
            pl.BlockSpec(memory_space=pltpu.SMEM),
        ],
        out_specs=pl.BlockSpec((1, _BLK), lambda i: (0, i)),
    )(flat2d, w1, b1, w2, b2)


def kernel(embeddings, W1, b1, W2, b2):
    bsz, seq, _ = embeddings.shape
    flat = embeddings.reshape(-1, _D)
    out = _run_tc(flat, W1, b1.reshape(_H, 1), W2.reshape(1, _H), b2)
    return out.reshape(bsz, seq)
